# baseline (speedup 1.0000x reference)
import functools

import jax
import jax.numpy as jnp
from jax import lax
from jax.experimental import pallas as pl
from jax.experimental.pallas import tpu as pltpu
from jax.experimental.pallas import tpu_sc as plsc

_NC = 2
_NS = 16
_NW = _NC * _NS
_CW = 512
_RING = 4


@functools.cache
def _gather_fn(B, D, V):
    n_chunks_total = (V + _CW - 1) // _CW
    n_rounds = (n_chunks_total + _NW - 1) // _NW
    last_c = n_chunks_total - 1
    last_w = V - last_c * _CW
    mesh = plsc.VectorSubcoreMesh(core_axis_name="c", subcore_axis_name="s")

    @functools.partial(
        pl.kernel,
        out_type=[
            jax.ShapeDtypeStruct((B + 8, 128), jnp.float32),
            jax.ShapeDtypeStruct((B + 8, 128), jnp.float32),
        ],
        mesh=mesh,
        scratch_types=[
            pltpu.VMEM((B,), jnp.int32),
            pltpu.VMEM((B + 16,), jnp.int32),
            pltpu.VMEM((B + 16,), jnp.int32),
            pltpu.VMEM((D, _CW), jnp.float32),
            pltpu.VMEM((D, _CW), jnp.float32),
            pltpu.VMEM((_RING, 16, 128), jnp.float32),
            pltpu.VMEM((_RING, 16), jnp.int32),
            pltpu.VMEM((D, 64), jnp.float32),
            pltpu.SemaphoreType.DMA,
            pltpu.SemaphoreType.DMA,
            pltpu.SemaphoreType.DMA,
        ],
        compiler_params=pltpu.CompilerParams(needs_layout_passes=False),
    )
    def gather(uids_hbm, utabT_hbm, utail_hbm, iids_hbm, itabT_hbm,
               itail_hbm, uout_hbm, iout_hbm,
               ids_v, my_ids, my_pos, buf0, buf1, rowbuf, posbuf, tailbuf,
               sem0, sem1, osem):
        wid = lax.axis_index("s") * _NC + lax.axis_index("c")
        kvec = lax.iota(jnp.int32, 16)
        dump = B + (kvec & 7)

        for ids_hbm, tabT_hbm, tail_hbm, out_hbm in (
                (uids_hbm, utabT_hbm, utail_hbm, uout_hbm),
                (iids_hbm, itabT_hbm, itail_hbm, iout_hbm)):
            pltpu.sync_copy(ids_hbm, ids_v)

            def filt(i, off):
                v = ids_v[pl.ds(i * 16, 16)]
                m = (lax.shift_right_logical(v, 9) & 31) == wid
                plsc.store_compressed(my_ids.at[pl.ds(off, 16)], v, mask=m)
                plsc.store_compressed(
                    my_pos.at[pl.ds(off, 16)], kvec + i * 16, mask=m)
                cnt = plsc.all_reduce_population_count(m)
                return off + cnt[0]

            n_mine = lax.fori_loop(0, B // 16, filt, 0)
            n_vregs = (n_mine + 15) // 16

            def fire(r, buf, sem):
                c = r * _NW + wid

                @pl.when(c < last_c)
                def _():
                    off = pl.multiple_of(c * _CW, 128)
                    pltpu.async_copy(
                        tabT_hbm.at[:, pl.ds(off, _CW)], buf, sem)

            def drain(r, buf, sem):
                c = r * _NW + wid

                @pl.when(c < last_c)
                def _():
                    pltpu.make_async_copy(
                        tabT_hbm.at[:, pl.ds(0, _CW)], buf, sem).wait()

            def extract(c, buf, mc_in):
                base_l = c * _CW

                def scan_vreg(j, mc):
                    v = my_ids[pl.ds(j * 16, 16)]
                    pos = my_pos[pl.ds(j * 16, 16)]
                    m = (lax.shift_right_logical(v, 9) == c) & (
                        kvec + j * 16 < n_mine)
                    hits = plsc.all_reduce_population_count(m)[0]

                    def do_extract():
                        slot = mc & (_RING - 1)

                        @pl.when(mc >= _RING)
                        def _():
                            pltpu.make_async_copy(
                                rowbuf.at[0], out_hbm.at[posbuf.at[0]],
                                osem).wait()

                        lvec = jnp.where(m, v - base_l, 0)
                        svec = jnp.full((16,), slot, jnp.int32)
                        for cd in range(D):
                            vals = plsc.load_gather(
                                buf, [jnp.full((16,), cd, jnp.int32), lvec])
                            plsc.store_scatter(
                                rowbuf, [svec, kvec, jnp.full(
                                    (16,), cd, jnp.int32)], vals)
                        posbuf[slot, pl.ds(0, 16)] = jnp.where(m, pos, dump)
                        pltpu.async_copy(
                            rowbuf.at[slot], out_hbm.at[posbuf.at[slot]],
                            osem)

                    return lax.cond(hits > 0,
                                    lambda: (do_extract(), mc + 1)[1],
                                    lambda: mc)

                return lax.fori_loop(0, n_vregs, scan_vreg, mc_in)

            fire(0, buf0, sem0)

            def rnd(i, mc):
                r0 = 2 * i
                fire(r0 + 1, buf1, sem1)
                drain(r0, buf0, sem0)
                mc = extract(r0 * _NW + wid, buf0, mc)

                @pl.when(r0 + 2 < n_rounds)
                def _():
                    fire(r0 + 2, buf0, sem0)

                drain(r0 + 1, buf1, sem1)
                return extract((r0 + 1) * _NW + wid, buf1, mc)

            mc = lax.fori_loop(0, n_rounds // 2, rnd, 0)

            def tail_extract():
                pltpu.sync_copy(tail_hbm, tailbuf)
                return extract(last_c, tailbuf, mc)

            mc = lax.cond(wid == last_c % _NW, tail_extract, lambda: mc)

            def final_drain(k, _):
                pltpu.make_async_copy(
                    rowbuf.at[0], out_hbm.at[posbuf.at[0]], osem).wait()
                return 0

            lax.fori_loop(0, jnp.minimum(mc, _RING), final_drain, 0)

    return gather


def _mlp_body(ur_ref, ir_ref, w1u_ref, w1i_ref, b1_ref, w2_ref, b2_ref,
              wo_ref, bo_ref, out_ref):
    D = 32
    ue = ur_ref[:, :D]
    ie = ir_ref[:, :D]
    x1 = jnp.dot(ue, w1u_ref[...], preferred_element_type=jnp.float32)
    x2 = jnp.dot(ie, w1i_ref[...], preferred_element_type=jnp.float32)
    h = jnp.maximum(x1 + x2 + b1_ref[...], 0.0)
    h = jnp.maximum(
        jnp.dot(h, w2_ref[...], preferred_element_type=jnp.float32)
        + b2_ref[...], 0.0)
    out_ref[...] = jnp.sum(h * wo_ref[...], axis=1) + bo_ref[0]


@functools.cache
def _mlp_fn(B, D, H1, H2, bb):
    grid = B // bb
    return pl.pallas_call(
        _mlp_body,
        grid=(grid,),
        in_specs=[
            pl.BlockSpec((bb, 128), lambda i: (i, 0)),
            pl.BlockSpec((bb, 128), lambda i: (i, 0)),
            pl.BlockSpec((D, H1), lambda i: (0, 0)),
            pl.BlockSpec((D, H1), lambda i: (0, 0)),
            pl.BlockSpec((1, H1), lambda i: (0, 0)),
            pl.BlockSpec((H1, H2), lambda i: (0, 0)),
            pl.BlockSpec((1, H2), lambda i: (0, 0)),
            pl.BlockSpec((1, H2), lambda i: (0, 0)),
            pl.BlockSpec((1,), lambda i: (0,)),
        ],
        out_specs=pl.BlockSpec((bb,), lambda i: (i,)),
        out_shape=jax.ShapeDtypeStruct((B,), jnp.float32),
    )


def kernel(user_ids, item_ids, user_table, item_table, W1, b1, W2, b2, Wo, bo):
    B = user_ids.shape[0]
    V, D = user_table.shape
    H1 = W1.shape[1]
    H2 = W2.shape[1]

    utabT = user_table.T
    itabT = item_table.T
    ur, ir = _gather_fn(B, D, V)(
        user_ids, utabT, utabT[:, V - 64:],
        item_ids, itabT, itabT[:, V - 64:])

    out = _mlp_fn(B, D, H1, H2, 512)(
        ur, ir, W1[:D], W1[D:], b1.reshape(1, H1), W2, b2.reshape(1, H2),
        Wo.reshape(1, H2), bo)
    return out


# speedup vs baseline: 8.6119x; 8.6119x over previous
import functools

import jax
import jax.numpy as jnp
from jax import lax
from jax.experimental import pallas as pl
from jax.experimental.pallas import tpu as pltpu
from jax.experimental.pallas import tpu_sc as plsc

_NC = 2
_NS = 16
_NW = _NC * _NS
_CW = 512
_RING = 16


@functools.cache
def _gather_fn(B, D, V):
    n_chunks_total = (V + _CW - 1) // _CW
    n_rounds = (n_chunks_total + _NW - 1) // _NW
    last_c = n_chunks_total - 1
    last_w = V - last_c * _CW
    n_pieces = 4
    piece = B // n_pieces
    mesh = plsc.VectorSubcoreMesh(core_axis_name="c", subcore_axis_name="s")

    @functools.partial(
        pl.kernel,
        out_type=[
            jax.ShapeDtypeStruct((B + 8, 128), jnp.float32),
            jax.ShapeDtypeStruct((B + 8, 128), jnp.float32),
        ],
        mesh=mesh,
        scratch_types=[
            pltpu.VMEM((piece,), jnp.int32),
            pltpu.VMEM((B + 16,), jnp.int32),
            pltpu.VMEM((D, _CW), jnp.float32),
            pltpu.VMEM((D, _CW), jnp.float32),
            pltpu.VMEM((_RING, 16, 128), jnp.float32),
            pltpu.VMEM((_RING, 16), jnp.int32),
            pltpu.VMEM((D, 64), jnp.float32),
            pltpu.SemaphoreType.DMA,
            pltpu.SemaphoreType.DMA,
            pltpu.SemaphoreType.DMA,
        ],
        compiler_params=pltpu.CompilerParams(needs_layout_passes=False),
    )
    def gather(uids_hbm, utabT_hbm, utail_hbm, iids_hbm, itabT_hbm,
               itail_hbm, uout_hbm, iout_hbm,
               idsbuf, my_keys, buf0, buf1, rowacc, posacc, tailbuf,
               sem0, sem1, osem):
        wid = lax.axis_index("s") * _NC + lax.axis_index("c")
        kvec = lax.iota(jnp.int32, 16)
        dump = B + (kvec & 7)

        for ids_hbm, tabT_hbm, tail_hbm, out_hbm in (
                (uids_hbm, utabT_hbm, utail_hbm, uout_hbm),
                (iids_hbm, itabT_hbm, itail_hbm, iout_hbm)):
            def filt_piece(p, off):
                pltpu.sync_copy(ids_hbm.at[pl.ds(p * piece, piece)], idsbuf)

                def filt(i, off):
                    v = idsbuf[pl.ds(i * 16, 16)]
                    m = (lax.shift_right_logical(v, 9) & 31) == wid
                    key = (
                        lax.shift_left(lax.shift_right_logical(v, 14), 23)
                        | lax.shift_left(v & 511, 14)
                        | (p * piece + i * 16 + kvec))
                    plsc.store_compressed(
                        my_keys.at[pl.ds(off, 16)], key, mask=m)
                    cnt = plsc.all_reduce_population_count(m)
                    return off + cnt[0]

                return lax.fori_loop(0, piece // 16, filt, off)

            n_mine = lax.fori_loop(0, n_pieces, filt_piece, 0)
            my_keys[pl.ds(n_mine, 16)] = jnp.full((16,), 1 << 30, jnp.int32)
            n_vregs = (n_mine + 15) // 16

            def fire(r, buf, sem):
                c = r * _NW + wid

                @pl.when(c < last_c)
                def _():
                    off = pl.multiple_of(c * _CW, 128)
                    pltpu.async_copy(
                        tabT_hbm.at[:, pl.ds(off, _CW)], buf, sem)

            def drain(r, buf, sem):
                c = r * _NW + wid

                @pl.when(c < last_c)
                def _():
                    pltpu.make_async_copy(
                        tabT_hbm.at[:, pl.ds(0, _CW)], buf, sem).wait()

            def out_fire(slot):
                pltpu.async_copy(
                    rowacc.at[slot], out_hbm.at[posacc.at[slot]], osem)

            def out_wait():
                pltpu.make_async_copy(
                    rowacc.at[0], out_hbm.at[posacc.at[0]], osem).wait()

            def extract(r, buf, acc_in):
                def scan_vreg(j, acc):
                    nacc, fired = acc
                    key = my_keys[pl.ds(j * 16, 16)]
                    m = lax.shift_right_logical(key, 23) == r
                    hits = plsc.all_reduce_population_count(m)[0]

                    def do_extract():
                        lvec = lax.shift_right_logical(key, 14) & 511
                        pos = key & 16383
                        rank = plsc.cumsum(jnp.where(m, 1, 0))
                        dest = nacc + rank - 1
                        dslot = lax.shift_right_logical(dest, 4) & (_RING - 1)
                        drow = dest & 15
                        for cd in range(D):
                            cdv = jnp.full((16,), cd, jnp.int32)
                            vals = plsc.load_gather(buf, [cdv, lvec],
                                                    mask=m)
                            plsc.store_scatter(rowacc, [dslot, drow, cdv],
                                               vals, mask=m)
                        plsc.store_scatter(posacc, [dslot, drow], pos,
                                           mask=m)
                        end = nacc + hits
                        done_slot = lax.shift_right_logical(nacc, 4)

                        def fire_done():
                            @pl.when(fired >= _RING - 2)
                            def _():
                                out_wait()
                            out_fire(done_slot & (_RING - 1))

                        return lax.cond(
                            lax.shift_right_logical(end, 4) > done_slot,
                            lambda: (fire_done(), (end, fired + 1))[1],
                            lambda: (end, fired))

                    return lax.cond(hits > 0, do_extract, lambda: acc)

                return lax.fori_loop(0, n_vregs, scan_vreg, acc_in)

            fire(0, buf0, sem0)

            def rnd(i, acc):
                r0 = 2 * i
                fire(r0 + 1, buf1, sem1)
                drain(r0, buf0, sem0)
                acc = extract(r0, buf0, acc)

                @pl.when(r0 + 2 < n_rounds)
                def _():
                    fire(r0 + 2, buf0, sem0)

                drain(r0 + 1, buf1, sem1)
                return extract(r0 + 1, buf1, acc)

            nacc, fired = lax.fori_loop(0, n_rounds // 2, rnd, (0, 0))

            def tail_extract():
                pltpu.sync_copy(tail_hbm, tailbuf)
                return extract(last_c // _NW, tailbuf, (nacc, fired))

            nacc, fired = lax.cond(
                wid == last_c % _NW, tail_extract, lambda: (nacc, fired))

            def flush():
                slot = lax.shift_right_logical(nacc, 4) & (_RING - 1)
                plsc.store_scatter(posacc, [jnp.full((16,), slot, jnp.int32),
                                            kvec], dump,
                                   mask=kvec >= (nacc & 15))
                out_fire(slot)
                return fired + 1

            fired = lax.cond((nacc & 15) > 0, flush, lambda: fired)

            def final_drain(k, _):
                out_wait()
                return 0

            lax.fori_loop(0, jnp.minimum(fired, _RING - 2), final_drain, 0)

    return gather


def _mlp_body(ur_ref, ir_ref, w1u_ref, w1i_ref, b1_ref, w2_ref, b2_ref,
              wo_ref, bo_ref, out_ref):
    D = 32
    ue = ur_ref[:, :D]
    ie = ir_ref[:, :D]
    x1 = jnp.dot(ue, w1u_ref[...], preferred_element_type=jnp.float32)
    x2 = jnp.dot(ie, w1i_ref[...], preferred_element_type=jnp.float32)
    h = jnp.maximum(x1 + x2 + b1_ref[...], 0.0)
    h = jnp.maximum(
        jnp.dot(h, w2_ref[...], preferred_element_type=jnp.float32)
        + b2_ref[...], 0.0)
    out_ref[...] = jnp.sum(h * wo_ref[...], axis=1) + bo_ref[0]


@functools.cache
def _mlp_fn(B, D, H1, H2, bb):
    grid = B // bb
    return pl.pallas_call(
        _mlp_body,
        grid=(grid,),
        in_specs=[
            pl.BlockSpec((bb, 128), lambda i: (i, 0)),
            pl.BlockSpec((bb, 128), lambda i: (i, 0)),
            pl.BlockSpec((D, H1), lambda i: (0, 0)),
            pl.BlockSpec((D, H1), lambda i: (0, 0)),
            pl.BlockSpec((1, H1), lambda i: (0, 0)),
            pl.BlockSpec((H1, H2), lambda i: (0, 0)),
            pl.BlockSpec((1, H2), lambda i: (0, 0)),
            pl.BlockSpec((1, H2), lambda i: (0, 0)),
            pl.BlockSpec((1,), lambda i: (0,)),
        ],
        out_specs=pl.BlockSpec((bb,), lambda i: (i,)),
        out_shape=jax.ShapeDtypeStruct((B,), jnp.float32),
    )


def kernel(user_ids, item_ids, user_table, item_table, W1, b1, W2, b2, Wo, bo):
    B = user_ids.shape[0]
    V, D = user_table.shape
    H1 = W1.shape[1]
    H2 = W2.shape[1]

    utabT = user_table.T
    itabT = item_table.T
    ur, ir = _gather_fn(B, D, V)(
        user_ids, utabT, utabT[:, V - 64:],
        item_ids, itabT, itabT[:, V - 64:])

    out = _mlp_fn(B, D, H1, H2, 512)(
        ur, ir, W1[:D], W1[D:], b1.reshape(1, H1), W2, b2.reshape(1, H2),
        Wo.reshape(1, H2), bo)
    return out


# speedup vs baseline: 10.4392x; 1.2122x over previous
import functools

import jax
import jax.numpy as jnp
from jax import lax
from jax.experimental import pallas as pl
from jax.experimental.pallas import tpu as pltpu
from jax.experimental.pallas import tpu_sc as plsc

_NC = 2
_NS = 16
_NW = _NC * _NS
_CW = 1024
_RING = 8
_LASTW = 576


@functools.cache
def _gather_fn(B, D, V):
    n_chunks_total = (V + _CW - 1) // _CW
    n_rounds = (n_chunks_total + _NW - 1) // _NW
    last_c = n_chunks_total - 1
    assert V - last_c * _CW == _LASTW
    n_pieces = 4
    piece = B // n_pieces
    mesh = plsc.VectorSubcoreMesh(core_axis_name="c", subcore_axis_name="s")

    @functools.partial(
        pl.kernel,
        out_type=[
            jax.ShapeDtypeStruct((B + 8, 128), jnp.float32),
            jax.ShapeDtypeStruct((B + 8, 128), jnp.float32),
        ],
        mesh=mesh,
        scratch_types=[
            pltpu.VMEM((piece,), jnp.int32),
            pltpu.VMEM((B + 16,), jnp.int32),
            pltpu.VMEM((D, _CW), jnp.float32),
            pltpu.VMEM((D, _CW), jnp.float32),
            pltpu.VMEM((_RING, 16, 128), jnp.float32),
            pltpu.VMEM((_RING, 16), jnp.int32),
            pltpu.VMEM((D, _LASTW), jnp.float32),
            pltpu.SemaphoreType.DMA,
            pltpu.SemaphoreType.DMA,
            pltpu.SemaphoreType.DMA,
        ],
        compiler_params=pltpu.CompilerParams(needs_layout_passes=False),
    )
    def gather(uids_hbm, utabT_hbm, utail_hbm, iids_hbm, itabT_hbm,
               itail_hbm, uout_hbm, iout_hbm,
               idsbuf, my_keys, buf0, buf1, rowacc, posacc, tailbuf,
               sem0, sem1, osem):
        wid = lax.axis_index("s") * _NC + lax.axis_index("c")
        kvec = lax.iota(jnp.int32, 16)
        dump = B + (kvec & 7)

        for ids_hbm, tabT_hbm, tail_hbm, out_hbm in (
                (uids_hbm, utabT_hbm, utail_hbm, uout_hbm),
                (iids_hbm, itabT_hbm, itail_hbm, iout_hbm)):
            def filt_piece(p, off):
                pltpu.sync_copy(ids_hbm.at[pl.ds(p * piece, piece)], idsbuf)

                def filt(i, off):
                    v = idsbuf[pl.ds(i * 16, 16)]
                    m = (lax.shift_right_logical(v, 10) & 31) == wid
                    key = (
                        lax.shift_left(lax.shift_right_logical(v, 15), 24)
                        | lax.shift_left(v & 1023, 14)
                        | (p * piece + i * 16 + kvec))
                    plsc.store_compressed(
                        my_keys.at[pl.ds(off, 16)], key, mask=m)
                    cnt = plsc.all_reduce_population_count(m)
                    return off + cnt[0]

                return lax.fori_loop(0, piece // 16, filt, off)

            n_mine = lax.fori_loop(0, n_pieces, filt_piece, 0)
            my_keys[pl.ds(n_mine, 16)] = jnp.full((16,), 1 << 30, jnp.int32)
            n_vregs = (n_mine + 15) // 16

            def fire(r, buf, sem):
                c = r * _NW + wid

                @pl.when(c < last_c)
                def _():
                    off = pl.multiple_of(c * _CW, 128)
                    pltpu.async_copy(
                        tabT_hbm.at[:, pl.ds(off, _CW)], buf, sem)

            def drain(r, buf, sem):
                c = r * _NW + wid

                @pl.when(c < last_c)
                def _():
                    pltpu.make_async_copy(
                        tabT_hbm.at[:, pl.ds(0, _CW)], buf, sem).wait()

            def out_fire(slot):
                pltpu.async_copy(
                    rowacc.at[slot], out_hbm.at[posacc.at[slot]], osem)

            def out_wait():
                pltpu.make_async_copy(
                    rowacc.at[0], out_hbm.at[posacc.at[0]], osem).wait()

            def extract(r, buf, acc_in):
                def scan_vreg(j, acc):
                    nacc, fired = acc
                    key = my_keys[pl.ds(j * 16, 16)]
                    m = lax.shift_right_logical(key, 24) == r
                    hits = plsc.all_reduce_population_count(m)[0]

                    def do_extract():
                        lvec = lax.shift_right_logical(key, 14) & 1023
                        pos = key & 16383
                        rank = plsc.cumsum(jnp.where(m, 1, 0))
                        dest = nacc + rank - 1
                        dslot = lax.shift_right_logical(dest, 4) & (_RING - 1)
                        drow = dest & 15
                        for cd in range(D):
                            cdv = jnp.full((16,), cd, jnp.int32)
                            vals = plsc.load_gather(buf, [cdv, lvec],
                                                    mask=m)
                            plsc.store_scatter(rowacc, [dslot, drow, cdv],
                                               vals, mask=m)
                        plsc.store_scatter(posacc, [dslot, drow], pos,
                                           mask=m)
                        end = nacc + hits
                        done_slot = lax.shift_right_logical(nacc, 4)

                        def fire_done():
                            @pl.when(fired >= _RING - 2)
                            def _():
                                out_wait()
                            out_fire(done_slot & (_RING - 1))

                        return lax.cond(
                            lax.shift_right_logical(end, 4) > done_slot,
                            lambda: (fire_done(), (end, fired + 1))[1],
                            lambda: (end, fired))

                    return lax.cond(hits > 0, do_extract, lambda: acc)

                return lax.fori_loop(0, n_vregs, scan_vreg, acc_in)

            fire(0, buf0, sem0)

            def guarded_extract(r, buf, acc):
                return lax.cond(r * _NW + wid < last_c,
                                lambda: extract(r, buf, acc),
                                lambda: acc)

            def rnd(i, acc):
                r0 = 2 * i
                fire(r0 + 1, buf1, sem1)
                drain(r0, buf0, sem0)
                acc = guarded_extract(r0, buf0, acc)

                @pl.when(r0 + 2 < n_rounds)
                def _():
                    fire(r0 + 2, buf0, sem0)

                drain(r0 + 1, buf1, sem1)
                return guarded_extract(r0 + 1, buf1, acc)

            acc = lax.fori_loop(0, n_rounds // 2, rnd, (0, 0))
            if n_rounds % 2:
                drain(n_rounds - 1, buf0, sem0)
                acc = guarded_extract(n_rounds - 1, buf0, acc)
            nacc, fired = acc

            def tail_extract():
                pltpu.sync_copy(tail_hbm, tailbuf)
                return extract(last_c // _NW, tailbuf, (nacc, fired))

            nacc, fired = lax.cond(
                wid == last_c % _NW, tail_extract, lambda: (nacc, fired))

            def flush():
                slot = lax.shift_right_logical(nacc, 4) & (_RING - 1)
                plsc.store_scatter(posacc, [jnp.full((16,), slot, jnp.int32),
                                            kvec], dump,
                                   mask=kvec >= (nacc & 15))
                out_fire(slot)
                return fired + 1

            fired = lax.cond((nacc & 15) > 0, flush, lambda: fired)

            def final_drain(k, _):
                out_wait()
                return 0

            lax.fori_loop(0, jnp.minimum(fired, _RING - 2), final_drain, 0)

    return gather


def _mlp_body(ur_ref, ir_ref, w1u_ref, w1i_ref, b1_ref, w2_ref, b2_ref,
              wo_ref, bo_ref, out_ref):
    D = 32
    ue = ur_ref[:, :D]
    ie = ir_ref[:, :D]
    x1 = jnp.dot(ue, w1u_ref[...], preferred_element_type=jnp.float32)
    x2 = jnp.dot(ie, w1i_ref[...], preferred_element_type=jnp.float32)
    h = jnp.maximum(x1 + x2 + b1_ref[...], 0.0)
    h = jnp.maximum(
        jnp.dot(h, w2_ref[...], preferred_element_type=jnp.float32)
        + b2_ref[...], 0.0)
    out_ref[...] = jnp.sum(h * wo_ref[...], axis=1) + bo_ref[0]


@functools.cache
def _mlp_fn(B, D, H1, H2, bb):
    grid = B // bb
    return pl.pallas_call(
        _mlp_body,
        grid=(grid,),
        in_specs=[
            pl.BlockSpec((bb, 128), lambda i: (i, 0)),
            pl.BlockSpec((bb, 128), lambda i: (i, 0)),
            pl.BlockSpec((D, H1), lambda i: (0, 0)),
            pl.BlockSpec((D, H1), lambda i: (0, 0)),
            pl.BlockSpec((1, H1), lambda i: (0, 0)),
            pl.BlockSpec((H1, H2), lambda i: (0, 0)),
            pl.BlockSpec((1, H2), lambda i: (0, 0)),
            pl.BlockSpec((1, H2), lambda i: (0, 0)),
            pl.BlockSpec((1,), lambda i: (0,)),
        ],
        out_specs=pl.BlockSpec((bb,), lambda i: (i,)),
        out_shape=jax.ShapeDtypeStruct((B,), jnp.float32),
    )


def kernel(user_ids, item_ids, user_table, item_table, W1, b1, W2, b2, Wo, bo):
    B = user_ids.shape[0]
    V, D = user_table.shape
    H1 = W1.shape[1]
    H2 = W2.shape[1]

    utabT = user_table.T
    itabT = item_table.T
    ur, ir = _gather_fn(B, D, V)(
        user_ids, utabT, utabT[:, V - _LASTW:],
        item_ids, itabT, itabT[:, V - _LASTW:])

    out = _mlp_fn(B, D, H1, H2, 512)(
        ur, ir, W1[:D], W1[D:], b1.reshape(1, H1), W2, b2.reshape(1, H2),
        Wo.reshape(1, H2), bo)
    return out


# speedup vs baseline: 12.5948x; 1.2065x over previous
import functools

import jax
import jax.numpy as jnp
from jax import lax
from jax.experimental import pallas as pl
from jax.experimental.pallas import tpu as pltpu
from jax.experimental.pallas import tpu_sc as plsc

_NC = 2
_NS = 16
_NW = _NC * _NS
_CW = 1024
_RING = 8
_LASTW = 576


@functools.cache
def _gather_fn(B, D, V):
    n_chunks_total = (V + _CW - 1) // _CW
    n_rounds = (n_chunks_total + _NW - 1) // _NW
    last_c = n_chunks_total - 1
    assert V - last_c * _CW == _LASTW
    n_pieces = 4
    piece = B // n_pieces
    mesh = plsc.VectorSubcoreMesh(core_axis_name="c", subcore_axis_name="s")

    @functools.partial(
        pl.kernel,
        out_type=[
            jax.ShapeDtypeStruct((B + 8, 128), jnp.float32),
            jax.ShapeDtypeStruct((B + 8, 128), jnp.float32),
        ],
        mesh=mesh,
        scratch_types=[
            pltpu.VMEM((piece,), jnp.int32),
            pltpu.VMEM((B + 16,), jnp.int32),
            pltpu.VMEM((D, _CW), jnp.float32),
            pltpu.VMEM((D, _CW), jnp.float32),
            pltpu.VMEM((_RING, 16, 128), jnp.float32),
            pltpu.VMEM((_RING, 16), jnp.int32),
            pltpu.VMEM((D, _LASTW), jnp.float32),
            pltpu.VMEM((272,), jnp.int32),
            pltpu.SemaphoreType.DMA,
            pltpu.SemaphoreType.DMA,
            pltpu.SemaphoreType.DMA,
        ],
        compiler_params=pltpu.CompilerParams(needs_layout_passes=False),
    )
    def gather(uids_hbm, utabT_hbm, utail_hbm, iids_hbm, itabT_hbm,
               itail_hbm, uout_hbm, iout_hbm,
               idsbuf, my_keys, buf0, buf1, rowacc, posacc, tailbuf,
               roundlist, sem0, sem1, osem):
        wid = lax.axis_index("s") * _NC + lax.axis_index("c")
        kvec = lax.iota(jnp.int32, 16)
        dump = B + (kvec & 7)

        for ids_hbm, tabT_hbm, tail_hbm, out_hbm in (
                (uids_hbm, utabT_hbm, utail_hbm, uout_hbm),
                (iids_hbm, itabT_hbm, itail_hbm, iout_hbm)):
            def filt_piece(p, off):
                pltpu.sync_copy(ids_hbm.at[pl.ds(p * piece, piece)], idsbuf)

                def filt(i, off):
                    v = idsbuf[pl.ds(i * 16, 16)]
                    m = (lax.shift_right_logical(v, 10) & 31) == wid
                    key = (
                        lax.shift_left(lax.shift_right_logical(v, 15), 24)
                        | lax.shift_left(v & 1023, 14)
                        | (p * piece + i * 16 + kvec))
                    plsc.store_compressed(
                        my_keys.at[pl.ds(off, 16)], key, mask=m)
                    cnt = plsc.all_reduce_population_count(m)
                    return off + cnt[0]

                return lax.fori_loop(0, piece // 16, filt, off)

            n_mine = lax.fori_loop(0, n_pieces, filt_piece, 0)
            my_keys[pl.ds(n_mine, 16)] = jnp.full((16,), 1 << 30, jnp.int32)
            n_vregs = (n_mine + 15) // 16

            def fire(r, buf, sem):
                c = r * _NW + wid

                @pl.when(c < last_c)
                def _():
                    off = pl.multiple_of(c * _CW, 128)
                    pltpu.async_copy(
                        tabT_hbm.at[:, pl.ds(off, _CW)], buf, sem)

            def drain(r, buf, sem):
                c = r * _NW + wid

                @pl.when(c < last_c)
                def _():
                    pltpu.make_async_copy(
                        tabT_hbm.at[:, pl.ds(0, _CW)], buf, sem).wait()

            def out_fire(slot):
                pltpu.async_copy(
                    rowacc.at[slot], out_hbm.at[posacc.at[slot]], osem)

            def out_wait():
                pltpu.make_async_copy(
                    rowacc.at[0], out_hbm.at[posacc.at[0]], osem).wait()

            def extract(r, buf, acc_in):
                def dense16(t_off, cnt, acc):
                    nacc, fired = acc
                    key = roundlist[pl.ds(t_off, 16)]
                    m = kvec < cnt
                    lvec = lax.shift_right_logical(key, 14) & 1023
                    pos = key & 16383
                    rank = plsc.cumsum(jnp.where(m, 1, 0))
                    dest = nacc + rank - 1
                    dslot = lax.shift_right_logical(dest, 4) & (_RING - 1)
                    drow = dest & 15
                    for cd in range(D):
                        cdv = jnp.full((16,), cd, jnp.int32)
                        vals = plsc.load_gather(buf, [cdv, lvec], mask=m)
                        plsc.store_scatter(rowacc, [dslot, drow, cdv],
                                           vals, mask=m)
                    plsc.store_scatter(posacc, [dslot, drow], pos, mask=m)
                    end = nacc + cnt
                    done_slot = lax.shift_right_logical(nacc, 4)

                    def fire_done():
                        @pl.when(fired >= _RING - 2)
                        def _():
                            out_wait()
                        out_fire(done_slot & (_RING - 1))

                    return lax.cond(
                        lax.shift_right_logical(end, 4) > done_slot,
                        lambda: (fire_done(), (end, fired + 1))[1],
                        lambda: (end, fired))

                def flush_full(rl, acc):
                    nfull = lax.shift_right_logical(rl, 4)
                    acc = lax.fori_loop(
                        0, nfull, lambda t, a: dense16(t * 16, 16, a), acc)
                    v = roundlist[pl.ds(nfull * 16, 16)]
                    roundlist[pl.ds(0, 16)] = v
                    return rl & 15, acc

                def scan_vreg(j, carry):
                    rl, nacc, fired = carry
                    key = my_keys[pl.ds(j * 16, 16)]
                    m = lax.shift_right_logical(key, 24) == r
                    hits = plsc.all_reduce_population_count(m)[0]

                    def compress():
                        plsc.store_compressed(
                            roundlist.at[pl.ds(rl, 16)], key, mask=m)
                        return rl + hits

                    rl = lax.cond(hits > 0, compress, lambda: rl)

                    def do_flush():
                        rl2, (na, fi) = flush_full(rl, (nacc, fired))
                        return rl2, na, fi

                    return lax.cond(rl >= 240, do_flush,
                                    lambda: (rl, nacc, fired))

                rl, nacc, fired = lax.fori_loop(
                    0, n_vregs, scan_vreg, (0,) + tuple(acc_in))
                rl, acc = flush_full(rl, (nacc, fired))
                return lax.cond(rl > 0,
                                lambda: dense16(0, rl, acc),
                                lambda: acc)

            fire(0, buf0, sem0)

            def guarded_extract(r, buf, acc):
                return lax.cond(r * _NW + wid < last_c,
                                lambda: extract(r, buf, acc),
                                lambda: acc)

            def rnd(i, acc):
                r0 = 2 * i
                fire(r0 + 1, buf1, sem1)
                drain(r0, buf0, sem0)
                acc = guarded_extract(r0, buf0, acc)

                @pl.when(r0 + 2 < n_rounds)
                def _():
                    fire(r0 + 2, buf0, sem0)

                drain(r0 + 1, buf1, sem1)
                return guarded_extract(r0 + 1, buf1, acc)

            acc = lax.fori_loop(0, n_rounds // 2, rnd, (0, 0))
            if n_rounds % 2:
                drain(n_rounds - 1, buf0, sem0)
                acc = guarded_extract(n_rounds - 1, buf0, acc)
            nacc, fired = acc

            def tail_extract():
                pltpu.sync_copy(tail_hbm, tailbuf)
                return extract(last_c // _NW, tailbuf, (nacc, fired))

            nacc, fired = lax.cond(
                wid == last_c % _NW, tail_extract, lambda: (nacc, fired))

            def flush():
                slot = lax.shift_right_logical(nacc, 4) & (_RING - 1)
                plsc.store_scatter(posacc, [jnp.full((16,), slot, jnp.int32),
                                            kvec], dump,
                                   mask=kvec >= (nacc & 15))
                out_fire(slot)
                return fired + 1

            fired = lax.cond((nacc & 15) > 0, flush, lambda: fired)

            def final_drain(k, _):
                out_wait()
                return 0

            lax.fori_loop(0, jnp.minimum(fired, _RING - 2), final_drain, 0)

    return gather


def _mlp_body(ur_ref, ir_ref, w1u_ref, w1i_ref, b1_ref, w2_ref, b2_ref,
              wo_ref, bo_ref, out_ref):
    D = 32
    ue = ur_ref[:, :D]
    ie = ir_ref[:, :D]
    x1 = jnp.dot(ue, w1u_ref[...], preferred_element_type=jnp.float32)
    x2 = jnp.dot(ie, w1i_ref[...], preferred_element_type=jnp.float32)
    h = jnp.maximum(x1 + x2 + b1_ref[...], 0.0)
    h = jnp.maximum(
        jnp.dot(h, w2_ref[...], preferred_element_type=jnp.float32)
        + b2_ref[...], 0.0)
    out_ref[...] = jnp.sum(h * wo_ref[...], axis=1) + bo_ref[0]


@functools.cache
def _mlp_fn(B, D, H1, H2, bb):
    grid = B // bb
    return pl.pallas_call(
        _mlp_body,
        grid=(grid,),
        in_specs=[
            pl.BlockSpec((bb, 128), lambda i: (i, 0)),
            pl.BlockSpec((bb, 128), lambda i: (i, 0)),
            pl.BlockSpec((D, H1), lambda i: (0, 0)),
            pl.BlockSpec((D, H1), lambda i: (0, 0)),
            pl.BlockSpec((1, H1), lambda i: (0, 0)),
            pl.BlockSpec((H1, H2), lambda i: (0, 0)),
            pl.BlockSpec((1, H2), lambda i: (0, 0)),
            pl.BlockSpec((1, H2), lambda i: (0, 0)),
            pl.BlockSpec((1,), lambda i: (0,)),
        ],
        out_specs=pl.BlockSpec((bb,), lambda i: (i,)),
        out_shape=jax.ShapeDtypeStruct((B,), jnp.float32),
    )


def kernel(user_ids, item_ids, user_table, item_table, W1, b1, W2, b2, Wo, bo):
    B = user_ids.shape[0]
    V, D = user_table.shape
    H1 = W1.shape[1]
    H2 = W2.shape[1]

    utabT = user_table.T
    itabT = item_table.T
    ur, ir = _gather_fn(B, D, V)(
        user_ids, utabT, utabT[:, V - _LASTW:],
        item_ids, itabT, itabT[:, V - _LASTW:])

    out = _mlp_fn(B, D, H1, H2, 512)(
        ur, ir, W1[:D], W1[D:], b1.reshape(1, H1), W2, b2.reshape(1, H2),
        Wo.reshape(1, H2), bo)
    return out


# speedup vs baseline: 12.7087x; 1.0090x over previous
import functools

import jax
import jax.numpy as jnp
from jax import lax
from jax.experimental import pallas as pl
from jax.experimental.pallas import tpu as pltpu
from jax.experimental.pallas import tpu_sc as plsc

_NC = 2
_NS = 16
_NW = _NC * _NS
_CW = 1024
_RING = 8
_LASTW = 576


@functools.cache
def _gather_fn(B, D, V):
    n_chunks_total = (V + _CW - 1) // _CW
    n_rounds = (n_chunks_total + _NW - 1) // _NW
    last_c = n_chunks_total - 1
    assert V - last_c * _CW == _LASTW
    n_pieces = 4
    piece = B // n_pieces
    mesh = plsc.VectorSubcoreMesh(core_axis_name="c", subcore_axis_name="s")

    @functools.partial(
        pl.kernel,
        out_type=[
            jax.ShapeDtypeStruct((B + 8, 128), jnp.float32),
            jax.ShapeDtypeStruct((B + 8, 128), jnp.float32),
        ],
        mesh=mesh,
        scratch_types=[
            pltpu.VMEM((piece,), jnp.int32),
            pltpu.VMEM((B + 16,), jnp.int32),
            pltpu.VMEM((D, _CW), jnp.float32),
            pltpu.VMEM((D, _CW), jnp.float32),
            pltpu.VMEM((_RING, 16, 128), jnp.float32),
            pltpu.VMEM((_RING, 16), jnp.int32),
            pltpu.VMEM((D, _LASTW), jnp.float32),
            pltpu.VMEM((272,), jnp.int32),
            pltpu.SemaphoreType.DMA,
            pltpu.SemaphoreType.DMA,
            pltpu.SemaphoreType.DMA,
        ],
        compiler_params=pltpu.CompilerParams(needs_layout_passes=False),
    )
    def gather(uids_hbm, utabT_hbm, utail_hbm, iids_hbm, itabT_hbm,
               itail_hbm, uout_hbm, iout_hbm,
               idsbuf, my_keys, buf0, buf1, rowacc, posacc, tailbuf,
               roundlist, sem0, sem1, osem):
        wid = lax.axis_index("s") * _NC + lax.axis_index("c")
        kvec = lax.iota(jnp.int32, 16)
        dump = B + (kvec & 7)

        for ids_hbm, tabT_hbm, tail_hbm, out_hbm in (
                (uids_hbm, utabT_hbm, utail_hbm, uout_hbm),
                (iids_hbm, itabT_hbm, itail_hbm, iout_hbm)):
            def filt_piece(p, off):
                pltpu.sync_copy(ids_hbm.at[pl.ds(p * piece, piece)], idsbuf)

                def filt(i, off):
                    v = idsbuf[pl.ds(i * 16, 16)]
                    m = (lax.shift_right_logical(v, 10) & 31) == wid
                    key = (
                        lax.shift_left(lax.shift_right_logical(v, 15), 24)
                        | lax.shift_left(v & 1023, 14)
                        | (p * piece + i * 16 + kvec))
                    plsc.store_compressed(
                        my_keys.at[pl.ds(off, 16)], key, mask=m)
                    cnt = plsc.all_reduce_population_count(m)
                    return off + cnt[0]

                return lax.fori_loop(0, piece // 16, filt, off)

            n_mine = lax.fori_loop(0, n_pieces, filt_piece, 0)
            my_keys[pl.ds(n_mine, 16)] = jnp.full((16,), 1 << 30, jnp.int32)
            n_vregs = (n_mine + 15) // 16

            def fire(r, buf, sem):
                c = r * _NW + wid

                @pl.when(c < last_c)
                def _():
                    off = pl.multiple_of(c * _CW, 128)
                    pltpu.async_copy(
                        tabT_hbm.at[:, pl.ds(off, _CW)], buf, sem)

            def drain(r, buf, sem):
                c = r * _NW + wid

                @pl.when(c < last_c)
                def _():
                    pltpu.make_async_copy(
                        tabT_hbm.at[:, pl.ds(0, _CW)], buf, sem).wait()

            def out_fire(slot):
                pltpu.async_copy(
                    rowacc.at[slot], out_hbm.at[posacc.at[slot]], osem)

            def out_wait():
                pltpu.make_async_copy(
                    rowacc.at[0], out_hbm.at[posacc.at[0]], osem).wait()

            def extract(r, buf, acc_in):
                def dense16(t_off, cnt, acc):
                    nacc, fired = acc
                    key = roundlist[pl.ds(t_off, 16)]
                    m = kvec < cnt
                    lvec = lax.shift_right_logical(key, 14) & 1023
                    pos = key & 16383
                    rank = plsc.cumsum(jnp.where(m, 1, 0))
                    dest = nacc + rank - 1
                    dslot = lax.shift_right_logical(dest, 4) & (_RING - 1)
                    drow = dest & 15
                    for cd in range(D):
                        cdv = jnp.full((16,), cd, jnp.int32)
                        vals = plsc.load_gather(buf, [cdv, lvec], mask=m)
                        plsc.store_scatter(rowacc, [dslot, drow, cdv],
                                           vals, mask=m)
                    plsc.store_scatter(posacc, [dslot, drow], pos, mask=m)
                    end = nacc + cnt
                    done_slot = lax.shift_right_logical(nacc, 4)

                    def fire_done():
                        @pl.when(fired >= _RING - 2)
                        def _():
                            out_wait()
                        out_fire(done_slot & (_RING - 1))

                    return lax.cond(
                        lax.shift_right_logical(end, 4) > done_slot,
                        lambda: (fire_done(), (end, fired + 1))[1],
                        lambda: (end, fired))

                def flush_full(rl, acc):
                    nfull = lax.shift_right_logical(rl, 4)
                    acc = lax.fori_loop(
                        0, nfull, lambda t, a: dense16(t * 16, 16, a), acc)
                    v = roundlist[pl.ds(nfull * 16, 16)]
                    roundlist[pl.ds(0, 16)] = v
                    return rl & 15, acc

                def scan_vreg(j, carry):
                    rl, nacc, fired = carry
                    key = my_keys[pl.ds(j * 16, 16)]
                    m = lax.shift_right_logical(key, 24) == r
                    hits = plsc.all_reduce_population_count(m)[0]

                    def compress():
                        plsc.store_compressed(
                            roundlist.at[pl.ds(rl, 16)], key, mask=m)
                        return rl + hits

                    rl = lax.cond(hits > 0, compress, lambda: rl)

                    def do_flush():
                        rl2, (na, fi) = flush_full(rl, (nacc, fired))
                        return rl2, na, fi

                    return lax.cond(rl >= 240, do_flush,
                                    lambda: (rl, nacc, fired))

                rl, nacc, fired = lax.fori_loop(
                    0, n_vregs, scan_vreg, (0,) + tuple(acc_in))
                rl, acc = flush_full(rl, (nacc, fired))
                return lax.cond(rl > 0,
                                lambda: dense16(0, rl, acc),
                                lambda: acc)

            fire(0, buf0, sem0)

            def guarded_extract(r, buf, acc):
                return lax.cond(r * _NW + wid < last_c,
                                lambda: extract(r, buf, acc),
                                lambda: acc)

            def rnd(i, acc):
                r0 = 2 * i
                fire(r0 + 1, buf1, sem1)
                drain(r0, buf0, sem0)
                acc = guarded_extract(r0, buf0, acc)

                @pl.when(r0 + 2 < n_rounds)
                def _():
                    fire(r0 + 2, buf0, sem0)

                drain(r0 + 1, buf1, sem1)
                return guarded_extract(r0 + 1, buf1, acc)

            acc = lax.fori_loop(0, n_rounds // 2, rnd, (0, 0))
            if n_rounds % 2:
                drain(n_rounds - 1, buf0, sem0)
                acc = guarded_extract(n_rounds - 1, buf0, acc)
            nacc, fired = acc

            def tail_extract():
                pltpu.sync_copy(tail_hbm, tailbuf)
                return extract(last_c // _NW, tailbuf, (nacc, fired))

            nacc, fired = lax.cond(
                wid == last_c % _NW, tail_extract, lambda: (nacc, fired))

            def flush():
                slot = lax.shift_right_logical(nacc, 4) & (_RING - 1)
                plsc.store_scatter(posacc, [jnp.full((16,), slot, jnp.int32),
                                            kvec], dump,
                                   mask=kvec >= (nacc & 15))
                out_fire(slot)
                return fired + 1

            fired = lax.cond((nacc & 15) > 0, flush, lambda: fired)

            def final_drain(k, _):
                out_wait()
                return 0

            lax.fori_loop(0, jnp.minimum(fired, _RING - 2), final_drain, 0)

    return gather


def _mlp_body(ur_ref, ir_ref, w1u_ref, w1i_ref, b1_ref, w2_ref, b2_ref,
              wo_ref, bo_ref, out_ref):
    D = 32
    ue = ur_ref[:, :D]
    ie = ir_ref[:, :D]
    x1 = jnp.dot(ue, w1u_ref[...], preferred_element_type=jnp.float32)
    x2 = jnp.dot(ie, w1i_ref[...], preferred_element_type=jnp.float32)
    h = jnp.maximum(x1 + x2 + b1_ref[...], 0.0)
    h = jnp.maximum(
        jnp.dot(h, w2_ref[...], preferred_element_type=jnp.float32)
        + b2_ref[...], 0.0)
    out_ref[...] = jnp.sum(h * wo_ref[...], axis=1) + bo_ref[0]


@functools.cache
def _mlp_fn(B, D, H1, H2, bb):
    grid = B // bb
    return pl.pallas_call(
        _mlp_body,
        grid=(grid,),
        in_specs=[
            pl.BlockSpec((bb, 128), lambda i: (i, 0)),
            pl.BlockSpec((bb, 128), lambda i: (i, 0)),
            pl.BlockSpec((D, H1), lambda i: (0, 0)),
            pl.BlockSpec((D, H1), lambda i: (0, 0)),
            pl.BlockSpec((1, H1), lambda i: (0, 0)),
            pl.BlockSpec((H1, H2), lambda i: (0, 0)),
            pl.BlockSpec((1, H2), lambda i: (0, 0)),
            pl.BlockSpec((1, H2), lambda i: (0, 0)),
            pl.BlockSpec((1,), lambda i: (0,)),
        ],
        out_specs=pl.BlockSpec((bb,), lambda i: (i,)),
        out_shape=jax.ShapeDtypeStruct((B,), jnp.float32),
    )


def kernel(user_ids, item_ids, user_table, item_table, W1, b1, W2, b2, Wo, bo):
    B = user_ids.shape[0]
    V, D = user_table.shape
    H1 = W1.shape[1]
    H2 = W2.shape[1]

    utabT = user_table.T
    itabT = item_table.T
    ur, ir = _gather_fn(B, D, V)(
        user_ids, utabT, utabT[:, V - _LASTW:],
        item_ids, itabT, itabT[:, V - _LASTW:])

    out = _mlp_fn(B, D, H1, H2, 2048)(
        ur, ir, W1[:D], W1[D:], b1.reshape(1, H1), W2, b2.reshape(1, H2),
        Wo.reshape(1, H2), bo)
    return out


# speedup vs baseline: 12.7928x; 1.0066x over previous
import functools

import jax
import jax.numpy as jnp
from jax import lax
from jax.experimental import pallas as pl
from jax.experimental.pallas import tpu as pltpu
from jax.experimental.pallas import tpu_sc as plsc

_NC = 2
_NS = 16
_NW = _NC * _NS
_CW = 1024
_RING = 8
_LASTW = 576


@functools.cache
def _gather_fn(B, D, V):
    n_chunks_total = (V + _CW - 1) // _CW
    n_rounds = (n_chunks_total + _NW - 1) // _NW
    last_c = n_chunks_total - 1
    assert V - last_c * _CW == _LASTW
    n_pieces = 4
    piece = B // n_pieces
    mesh = plsc.VectorSubcoreMesh(core_axis_name="c", subcore_axis_name="s")

    @functools.partial(
        pl.kernel,
        out_type=[
            jax.ShapeDtypeStruct((B + 8, 128), jnp.float32),
            jax.ShapeDtypeStruct((B + 8, 128), jnp.float32),
        ],
        mesh=mesh,
        scratch_types=[
            pltpu.VMEM((piece,), jnp.int32),
            pltpu.VMEM((B + 16,), jnp.int32),
            pltpu.VMEM((D, _CW), jnp.float32),
            pltpu.VMEM((D, _CW), jnp.float32),
            pltpu.VMEM((_RING, 16, 128), jnp.float32),
            pltpu.VMEM((_RING, 16), jnp.int32),
            pltpu.VMEM((D, _LASTW), jnp.float32),
            pltpu.VMEM((272,), jnp.int32),
            pltpu.SemaphoreType.DMA,
            pltpu.SemaphoreType.DMA,
            pltpu.SemaphoreType.DMA,
        ],
        compiler_params=pltpu.CompilerParams(needs_layout_passes=False),
    )
    def gather(uids_hbm, utabT_hbm, utail_hbm, iids_hbm, itabT_hbm,
               itail_hbm, uout_hbm, iout_hbm,
               idsbuf, my_keys, buf0, buf1, rowacc, posacc, tailbuf,
               roundlist, sem0, sem1, osem):
        wid = lax.axis_index("s") * _NC + lax.axis_index("c")
        kvec = lax.iota(jnp.int32, 16)
        dump = B + (kvec & 7)

        for ids_hbm, tabT_hbm, tail_hbm, out_hbm in (
                (uids_hbm, utabT_hbm, utail_hbm, uout_hbm),
                (iids_hbm, itabT_hbm, itail_hbm, iout_hbm)):
            pltpu.async_copy(
                tabT_hbm.at[:, pl.ds(pl.multiple_of(wid * _CW, 128), _CW)],
                buf0, sem0)
            pltpu.async_copy(
                tabT_hbm.at[:, pl.ds(
                    pl.multiple_of((_NW + wid) * _CW, 128), _CW)],
                buf1, sem1)

            def filt_piece(p, off):
                pltpu.sync_copy(ids_hbm.at[pl.ds(p * piece, piece)], idsbuf)

                def filt2(i, off):
                    v0 = idsbuf[pl.ds(i * 32, 16)]
                    v1 = idsbuf[pl.ds(i * 32 + 16, 16)]
                    m0 = (lax.shift_right_logical(v0, 10) & 31) == wid
                    m1 = (lax.shift_right_logical(v1, 10) & 31) == wid
                    k0 = (
                        lax.shift_left(lax.shift_right_logical(v0, 15), 24)
                        | lax.shift_left(v0 & 1023, 14)
                        | (p * piece + i * 32 + kvec))
                    k1 = (
                        lax.shift_left(lax.shift_right_logical(v1, 15), 24)
                        | lax.shift_left(v1 & 1023, 14)
                        | (p * piece + i * 32 + 16 + kvec))
                    c0 = plsc.all_reduce_population_count(m0)[0]
                    c1 = plsc.all_reduce_population_count(m1)[0]
                    plsc.store_compressed(
                        my_keys.at[pl.ds(off, 16)], k0, mask=m0)
                    plsc.store_compressed(
                        my_keys.at[pl.ds(off + c0, 16)], k1, mask=m1)
                    return off + c0 + c1

                return lax.fori_loop(0, piece // 32, filt2, off)

            n_mine = lax.fori_loop(0, n_pieces, filt_piece, 0)
            my_keys[pl.ds(n_mine, 16)] = jnp.full((16,), 1 << 30, jnp.int32)
            n_vregs = (n_mine + 15) // 16

            def fire(r, buf, sem):
                c = r * _NW + wid

                @pl.when(c < last_c)
                def _():
                    off = pl.multiple_of(c * _CW, 128)
                    pltpu.async_copy(
                        tabT_hbm.at[:, pl.ds(off, _CW)], buf, sem)

            def drain(r, buf, sem):
                c = r * _NW + wid

                @pl.when(c < last_c)
                def _():
                    pltpu.make_async_copy(
                        tabT_hbm.at[:, pl.ds(0, _CW)], buf, sem).wait()

            def out_fire(slot):
                pltpu.async_copy(
                    rowacc.at[slot], out_hbm.at[posacc.at[slot]], osem)

            def out_wait():
                pltpu.make_async_copy(
                    rowacc.at[0], out_hbm.at[posacc.at[0]], osem).wait()

            def extract(r, buf, acc_in):
                def dense16(t_off, cnt, acc):
                    nacc, fired = acc
                    key = roundlist[pl.ds(t_off, 16)]
                    m = kvec < cnt
                    lvec = lax.shift_right_logical(key, 14) & 1023
                    pos = key & 16383
                    rank = plsc.cumsum(jnp.where(m, 1, 0))
                    dest = nacc + rank - 1
                    dslot = lax.shift_right_logical(dest, 4) & (_RING - 1)
                    drow = dest & 15
                    for cd in range(D):
                        cdv = jnp.full((16,), cd, jnp.int32)
                        vals = plsc.load_gather(buf, [cdv, lvec], mask=m)
                        plsc.store_scatter(rowacc, [dslot, drow, cdv],
                                           vals, mask=m)
                    plsc.store_scatter(posacc, [dslot, drow], pos, mask=m)
                    end = nacc + cnt
                    done_slot = lax.shift_right_logical(nacc, 4)

                    def fire_done():
                        @pl.when(fired >= _RING - 2)
                        def _():
                            out_wait()
                        out_fire(done_slot & (_RING - 1))

                    return lax.cond(
                        lax.shift_right_logical(end, 4) > done_slot,
                        lambda: (fire_done(), (end, fired + 1))[1],
                        lambda: (end, fired))

                def flush_full(rl, acc):
                    nfull = lax.shift_right_logical(rl, 4)
                    acc = lax.fori_loop(
                        0, nfull, lambda t, a: dense16(t * 16, 16, a), acc)
                    v = roundlist[pl.ds(nfull * 16, 16)]
                    roundlist[pl.ds(0, 16)] = v
                    return rl & 15, acc

                def scan_vreg(j, carry):
                    rl, nacc, fired = carry
                    key = my_keys[pl.ds(j * 16, 16)]
                    m = lax.shift_right_logical(key, 24) == r
                    hits = plsc.all_reduce_population_count(m)[0]

                    def compress():
                        plsc.store_compressed(
                            roundlist.at[pl.ds(rl, 16)], key, mask=m)
                        return rl + hits

                    rl = lax.cond(hits > 0, compress, lambda: rl)

                    def do_flush():
                        rl2, (na, fi) = flush_full(rl, (nacc, fired))
                        return rl2, na, fi

                    return lax.cond(rl >= 240, do_flush,
                                    lambda: (rl, nacc, fired))

                rl, nacc, fired = lax.fori_loop(
                    0, n_vregs, scan_vreg, (0,) + tuple(acc_in))
                rl, acc = flush_full(rl, (nacc, fired))
                return lax.cond(rl > 0,
                                lambda: dense16(0, rl, acc),
                                lambda: acc)

            def guarded_extract(r, buf, acc):
                return lax.cond(r * _NW + wid < last_c,
                                lambda: extract(r, buf, acc),
                                lambda: acc)

            def rnd(i, acc):
                r0 = 2 * i
                drain(r0, buf0, sem0)
                acc = guarded_extract(r0, buf0, acc)

                @pl.when(r0 + 2 < n_rounds)
                def _():
                    fire(r0 + 2, buf0, sem0)

                drain(r0 + 1, buf1, sem1)
                acc = guarded_extract(r0 + 1, buf1, acc)

                @pl.when(r0 + 3 < n_rounds)
                def _():
                    fire(r0 + 3, buf1, sem1)

                return acc

            acc = lax.fori_loop(0, n_rounds // 2, rnd, (0, 0))
            if n_rounds % 2:
                drain(n_rounds - 1, buf0, sem0)
                acc = guarded_extract(n_rounds - 1, buf0, acc)
            nacc, fired = acc

            def tail_extract():
                pltpu.sync_copy(tail_hbm, tailbuf)
                return extract(last_c // _NW, tailbuf, (nacc, fired))

            nacc, fired = lax.cond(
                wid == last_c % _NW, tail_extract, lambda: (nacc, fired))

            def flush():
                slot = lax.shift_right_logical(nacc, 4) & (_RING - 1)
                plsc.store_scatter(posacc, [jnp.full((16,), slot, jnp.int32),
                                            kvec], dump,
                                   mask=kvec >= (nacc & 15))
                out_fire(slot)
                return fired + 1

            fired = lax.cond((nacc & 15) > 0, flush, lambda: fired)

            def final_drain(k, _):
                out_wait()
                return 0

            lax.fori_loop(0, jnp.minimum(fired, _RING - 2), final_drain, 0)

    return gather


def _mlp_body(ur_ref, ir_ref, w1u_ref, w1i_ref, b1_ref, w2_ref, b2_ref,
              wo_ref, bo_ref, out_ref):
    D = 32
    ue = ur_ref[:, :D]
    ie = ir_ref[:, :D]
    x1 = jnp.dot(ue, w1u_ref[...], preferred_element_type=jnp.float32)
    x2 = jnp.dot(ie, w1i_ref[...], preferred_element_type=jnp.float32)
    h = jnp.maximum(x1 + x2 + b1_ref[...], 0.0)
    h = jnp.maximum(
        jnp.dot(h, w2_ref[...], preferred_element_type=jnp.float32)
        + b2_ref[...], 0.0)
    out_ref[...] = jnp.sum(h * wo_ref[...], axis=1) + bo_ref[0]


@functools.cache
def _mlp_fn(B, D, H1, H2, bb):
    grid = B // bb
    return pl.pallas_call(
        _mlp_body,
        grid=(grid,),
        in_specs=[
            pl.BlockSpec((bb, 128), lambda i: (i, 0)),
            pl.BlockSpec((bb, 128), lambda i: (i, 0)),
            pl.BlockSpec((D, H1), lambda i: (0, 0)),
            pl.BlockSpec((D, H1), lambda i: (0, 0)),
            pl.BlockSpec((1, H1), lambda i: (0, 0)),
            pl.BlockSpec((H1, H2), lambda i: (0, 0)),
            pl.BlockSpec((1, H2), lambda i: (0, 0)),
            pl.BlockSpec((1, H2), lambda i: (0, 0)),
            pl.BlockSpec((1,), lambda i: (0,)),
        ],
        out_specs=pl.BlockSpec((bb,), lambda i: (i,)),
        out_shape=jax.ShapeDtypeStruct((B,), jnp.float32),
    )


def kernel(user_ids, item_ids, user_table, item_table, W1, b1, W2, b2, Wo, bo):
    B = user_ids.shape[0]
    V, D = user_table.shape
    H1 = W1.shape[1]
    H2 = W2.shape[1]

    utabT = user_table.T
    itabT = item_table.T
    ur, ir = _gather_fn(B, D, V)(
        user_ids, utabT, utabT[:, V - _LASTW:],
        item_ids, itabT, itabT[:, V - _LASTW:])

    out = _mlp_fn(B, D, H1, H2, 2048)(
        ur, ir, W1[:D], W1[D:], b1.reshape(1, H1), W2, b2.reshape(1, H2),
        Wo.reshape(1, H2), bo)
    return out
